# Initial kernel scaffold; baseline (speedup 1.0000x reference)
import jax
import jax.numpy as jnp
from jax.experimental import pallas as pl


def kernel(x):
    raise NotImplementedError("write your pallas kernel here")


# speedup vs baseline: 1.4828x; 1.4828x over previous
import functools
import math

import jax
import jax.numpy as jnp
from jax import lax
from jax.experimental import pallas as pl
from jax.experimental.pallas import tpu as pltpu
from jax.experimental.pallas import tpu_sc as plsc

_B, _S, _D = 16, 4096, 1024
_TARGET = 512
_DIFF = _S % _TARGET
_STEP = math.ceil((_S - _DIFF) / _TARGET)
_OFF = _DIFF // 2

_R = _B * _TARGET
_NW = 32
_RPW = _R // _NW
_CHUNK = 64
_NCHUNK = _RPW // _CHUNK

_mesh = plsc.VectorSubcoreMesh(core_axis_name="c", subcore_axis_name="s")


@functools.partial(
    pl.kernel,
    mesh=_mesh,
    out_type=jax.ShapeDtypeStruct((_R, _D), jnp.float32),
    scratch_types=[
        pltpu.VMEM((_CHUNK,), jnp.int32),
        pltpu.VMEM((_CHUNK, _D), jnp.float32),
        pltpu.SemaphoreType.DMA,
    ],
)
def _downsample(x_hbm, idx_hbm, out_hbm, idx_v, rows_v, sem):
    wid = lax.axis_index("s") * 2 + lax.axis_index("c")
    base = wid * _RPW

    def body(g, carry):
        rb = base + g * _CHUNK
        pltpu.sync_copy(idx_hbm.at[pl.ds(rb, _CHUNK)], idx_v)
        pltpu.async_copy(x_hbm.at[idx_v], rows_v, sem).wait()
        pltpu.sync_copy(rows_v, out_hbm.at[pl.ds(rb, _CHUNK)])
        return carry

    lax.fori_loop(0, _NCHUNK, body, 0)


def kernel(x):
    xf = x.reshape(_B * _S, _D)
    idx = jnp.arange(_R, dtype=jnp.int32) * _STEP
    idx = idx + _OFF
    out = _downsample(xf, idx)
    return out.reshape(_B, _TARGET, _D)


# speedup vs baseline: 1.5192x; 1.0245x over previous
import functools
import math

import jax
import jax.numpy as jnp
from jax import lax
from jax.experimental import pallas as pl
from jax.experimental.pallas import tpu as pltpu
from jax.experimental.pallas import tpu_sc as plsc

_B, _S, _D = 16, 4096, 1024
_TARGET = 512
_DIFF = _S % _TARGET
_STEP = math.ceil((_S - _DIFF) / _TARGET)
_OFF = _DIFF // 2

_R = _B * _TARGET
_NW = 32
_RPW = _R // _NW
_CHUNK = 32
_NCHUNK = _RPW // _CHUNK

_mesh = plsc.VectorSubcoreMesh(core_axis_name="c", subcore_axis_name="s")


@functools.partial(
    pl.kernel,
    mesh=_mesh,
    out_type=jax.ShapeDtypeStruct((_R, _D), jnp.float32),
    scratch_types=[
        pltpu.VMEM((_RPW,), jnp.int32),
        pltpu.VMEM((_CHUNK, _D), jnp.float32),
        pltpu.VMEM((_CHUNK, _D), jnp.float32),
        pltpu.SemaphoreType.DMA,
        pltpu.SemaphoreType.DMA,
        pltpu.SemaphoreType.DMA,
        pltpu.SemaphoreType.DMA,
    ],
)
def _downsample(x_hbm, idx_hbm, out_hbm, idx_v, rows0, rows1,
                gsem0, gsem1, ssem0, ssem1):
    wid = lax.axis_index("s") * 2 + lax.axis_index("c")
    base = wid * _RPW
    pltpu.sync_copy(idx_hbm.at[pl.ds(base, _RPW)], idx_v)

    bufs = (rows0, rows1)
    gsems = (gsem0, gsem1)
    ssems = (ssem0, ssem1)

    def gather(g):
        return pltpu.async_copy(
            x_hbm.at[idx_v.at[pl.ds(g * _CHUNK, _CHUNK)]],
            bufs[g % 2], gsems[g % 2])

    def scatter(g):
        return pltpu.async_copy(
            bufs[g % 2], out_hbm.at[pl.ds(base + g * _CHUNK, _CHUNK)],
            ssems[g % 2])

    gathers = [None] * _NCHUNK
    scatters = [None] * _NCHUNK
    gathers[0] = gather(0)
    for g in range(_NCHUNK):
        if g + 1 < _NCHUNK:
            if g + 1 >= 2:
                scatters[g - 1].wait()
            gathers[g + 1] = gather(g + 1)
        gathers[g].wait()
        scatters[g] = scatter(g)
    scatters[_NCHUNK - 2].wait()
    scatters[_NCHUNK - 1].wait()


def kernel(x):
    xf = x.reshape(_B * _S, _D)
    idx = jnp.arange(_R, dtype=jnp.int32) * _STEP + _OFF
    out = _downsample(xf, idx)
    return out.reshape(_B, _TARGET, _D)
